# baseline (speedup 1.0000x reference)
import functools

import jax
import jax.numpy as jnp
from jax import lax
from jax.experimental import pallas as pl
from jax.experimental.pallas import tpu as pltpu
from jax.experimental.pallas import tpu_sc as plsc

EMB_SIZE = 32
PITCH = 33

_info = plsc.get_sparse_core_info()
_NC, _NS = _info.num_cores, _info.num_subcores
_NW = _NC * _NS


def _make_gather(B: int, D: int, chunk: int, nbuf: int = 4, dist: int = 2):
    assert B % (_NW * chunk) == 0
    b_per_w = B // _NW
    n_chunks = b_per_w // chunk
    assert n_chunks % nbuf == 0 and dist < nbuf
    mesh = plsc.VectorSubcoreMesh(core_axis_name="c", subcore_axis_name="s")

    @functools.partial(
        pl.kernel,
        out_type=jax.ShapeDtypeStruct((B, D), jnp.float32),
        mesh=mesh,
        scratch_types=[
            pltpu.VMEM((b_per_w,), jnp.int32),
            *[pltpu.VMEM((chunk, D), jnp.float32) for _ in range(nbuf)],
            *[pltpu.SemaphoreType.DMA for _ in range(2 * nbuf)],
        ],
        compiler_params=pltpu.CompilerParams(use_tc_tiling_on_sc=False),
    )
    def gather_kernel(idx_hbm, table_hbm, out_hbm, idx_v, *scratch):
        rows = scratch[:nbuf]
        gsem = scratch[nbuf : 2 * nbuf]
        osem = scratch[2 * nbuf :]
        wid = lax.axis_index("s") * _NC + lax.axis_index("c")
        base = wid * b_per_w
        pltpu.sync_copy(idx_hbm.at[pl.ds(base, b_per_w)], idx_v)

        def g_start(i, b):
            pltpu.async_copy(
                table_hbm.at[idx_v.at[pl.ds(i * chunk, chunk)]], rows[b], gsem[b]
            )

        def g_wait(i, b):
            pltpu.make_async_copy(
                table_hbm.at[idx_v.at[pl.ds(i * chunk, chunk)]], rows[b], gsem[b]
            ).wait()

        def o_start(i, b):
            pltpu.async_copy(
                rows[b], out_hbm.at[pl.ds(base + i * chunk, chunk)], osem[b]
            )

        def o_wait(i, b):
            pltpu.make_async_copy(
                rows[b], out_hbm.at[pl.ds(base + i * chunk, chunk)], osem[b]
            ).wait()

        for b in range(dist):
            g_start(b, b)

        def body(g, carry):
            for b in range(nbuf):
                i = g * nbuf + b
                g_wait(i, b)
                o_start(i, b)
                nxt = i + dist
                c = (b + dist) % nbuf

                @pl.when(nxt < n_chunks)
                def _():
                    @pl.when(nxt >= nbuf)
                    def _():
                        o_wait(nxt - nbuf, c)

                    g_start(nxt, c)

            return carry

        lax.fori_loop(0, n_chunks // nbuf, body, 0)
        for b in range(nbuf):
            o_wait(n_chunks - nbuf + b, b)

    return gather_kernel


def _make_detile(V: int, D: int, n: int, m: int):
    assert D == 32 and V == 100000 and n == 16384 and m == 50
    full_tiles = V // 128
    rem = V - full_tiles * 128
    per_w = full_tiles // _NW + 1
    xL = 4096
    x_items_full = (m // 8) * (n // xL)
    x_rem_rows = m - (m // 8) * 8
    mesh = plsc.VectorSubcoreMesh(core_axis_name="c", subcore_axis_name="s")

    @functools.partial(
        pl.kernel,
        out_type=[
            jax.ShapeDtypeStruct((V * PITCH,), jnp.float32),
            jax.ShapeDtypeStruct((m * n,), jnp.int32),
        ],
        mesh=mesh,
        scratch_types=[
            pltpu.VMEM((D, 128), jnp.float32),
            pltpu.VMEM((D, rem), jnp.float32),
            pltpu.VMEM((128 * PITCH,), jnp.float32),
            pltpu.VMEM((rem * PITCH,), jnp.float32),
            pltpu.VMEM((8, xL), jnp.int32),
            pltpu.VMEM((x_rem_rows, xL), jnp.int32),
        ],
        compiler_params=pltpu.CompilerParams(
            use_tc_tiling_on_sc=True, needs_layout_passes=False
        ),
    )
    def detile_kernel(tab_hbm, x_hbm, tab_out, idx_out, tblk, tblk_p, rowbuf, rowbuf_p, xblk8, xblk2):
        wid = lax.axis_index("s") * _NC + lax.axis_index("c")
        iota = lax.iota(jnp.int32, 16)
        P = [(iota + l0) * PITCH for l0 in range(0, 128, 16)]

        def detile_block(src, dst, width):
            def erow(e, carry):
                for li in range(width // 16):
                    vals = src[e, pl.ds(li * 16, 16)]
                    plsc.store_scatter(dst, [P[li] + e], vals)
                return carry

            lax.fori_loop(0, D, erow, 0)

        def tblock(t, carry):
            k = wid + t * _NW

            @pl.when(k < full_tiles)
            def _():
                pltpu.sync_copy(tab_hbm.at[:, pl.ds(k * 128, 128)], tblk)
                detile_block(tblk, rowbuf, 128)
                pltpu.sync_copy(
                    rowbuf, tab_out.at[pl.ds(k * 128 * PITCH, 128 * PITCH)]
                )

            return carry

        lax.fori_loop(0, per_w, tblock, 0)

        @pl.when(wid == full_tiles % _NW)
        def _():
            v0 = full_tiles * 128
            pltpu.sync_copy(tab_hbm.at[:, pl.ds(v0, rem)], tblk_p)
            detile_block(tblk_p, rowbuf_p, rem)
            pltpu.sync_copy(rowbuf_p, tab_out.at[pl.ds(v0 * PITCH, rem * PITCH)])

        @pl.when(wid < x_items_full)
        def _():
            jr = wid // (n // xL)
            tcb = wid % (n // xL)
            pltpu.sync_copy(
                x_hbm.at[pl.ds(jr * 8, 8), pl.ds(tcb * xL, xL)], xblk8
            )
            for s in range(8):
                pltpu.sync_copy(
                    xblk8.at[s],
                    idx_out.at[pl.ds((jr * 8 + s) * n + tcb * xL, xL)],
                )

        @pl.when((wid >= x_items_full) & (wid < x_items_full + n // xL))
        def _():
            tcb = wid - x_items_full
            j0 = (m // 8) * 8
            pltpu.sync_copy(
                x_hbm.at[pl.ds(j0, x_rem_rows), pl.ds(tcb * xL, xL)], xblk2
            )
            for s in range(x_rem_rows):
                pltpu.sync_copy(
                    xblk2.at[s],
                    idx_out.at[pl.ds((j0 + s) * n + tcb * xL, xL)],
                )

    return detile_kernel


def _make_retile(m: int, n: int, D: int):
    IB = 512
    assert n == _NW * IB and D % 8 == 0
    ntr = D // 8
    mesh = plsc.VectorSubcoreMesh(core_axis_name="c", subcore_axis_name="s")

    @functools.partial(
        pl.kernel,
        out_type=jax.ShapeDtypeStruct((m, D, n), jnp.float32),
        mesh=mesh,
        scratch_types=[
            *[pltpu.VMEM((IB * PITCH,), jnp.float32) for _ in range(2)],
            *[pltpu.VMEM((ntr, 8, IB), jnp.float32) for _ in range(2)],
            *[pltpu.SemaphoreType.DMA for _ in range(4)],
        ],
        compiler_params=pltpu.CompilerParams(
            use_tc_tiling_on_sc=True, needs_layout_passes=False
        ),
    )
    def retile_kernel(in_hbm, out_hbm, inb0, inb1, tb0, tb1, is0, is1, os0, os1):
        inb = (inb0, inb1)
        tb = (tb0, tb1)
        isem = (is0, is1)
        osem = (os0, os1)
        wid = lax.axis_index("s") * _NC + lax.axis_index("c")
        iota = lax.iota(jnp.int32, 16)

        def in_off(j):
            return (j * n + wid * IB) * PITCH

        def i_start(j, p):
            pltpu.async_copy(in_hbm.at[pl.ds(in_off(j), IB * PITCH)], inb[p], isem[p])

        def i_wait(j, p):
            pltpu.make_async_copy(
                in_hbm.at[pl.ds(in_off(j), IB * PITCH)], inb[p], isem[p]
            ).wait()

        def o_start(j, p):
            for tr in range(ntr):
                pltpu.async_copy(
                    tb[p].at[tr],
                    out_hbm.at[j, pl.ds(tr * 8, 8), pl.ds(wid * IB, IB)],
                    osem[p],
                )

        def o_wait(j, p):
            for tr in range(ntr):
                pltpu.make_async_copy(
                    tb[p].at[tr],
                    out_hbm.at[j, pl.ds(tr * 8, 8), pl.ds(wid * IB, IB)],
                    osem[p],
                ).wait()

        i_start(0, 0)

        def body(jg, carry):
            for p in range(2):
                j = 2 * jg + p
                i_wait(j, p)

                @pl.when(j + 1 < m)
                def _():
                    i_start(j + 1, 1 - p)

                @pl.when(j >= 2)
                def _():
                    o_wait(j - 2, p)

                @plsc.parallel_loop(0, IB // 16, unroll=4)
                def lgroup(l0g):
                    src = (iota + l0g * 16) * PITCH
                    l0 = l0g * 16
                    for e in range(D):
                        vals = plsc.load_gather(inb[p], [src + e])
                        tb[p][e // 8, e % 8, pl.ds(l0, 16)] = vals

                o_start(j, p)

            return carry

        lax.fori_loop(0, m // 2, body, 0)
        o_wait(m - 2, 0)
        o_wait(m - 1, 1)

    return retile_kernel


def kernel(x, atom_emb_weight):
    n, m = x.shape
    B = x.size
    V, D = atom_emb_weight.shape
    tab1d, idx1d = _make_detile(V, D, n, m)(
        atom_emb_weight.T, x.T.astype(jnp.int32)
    )
    out = _make_gather(B, PITCH, 640)(idx1d, tab1d.reshape(V, PITCH))
    out_t = _make_retile(m, n, D)(out.reshape(-1))
    return out_t.transpose(2, 0, 1)


# speedup vs baseline: 3.2125x; 3.2125x over previous
import functools

import jax
import jax.numpy as jnp
from jax import lax
from jax.experimental import pallas as pl
from jax.experimental.pallas import tpu as pltpu
from jax.experimental.pallas import tpu_sc as plsc

EMB_SIZE = 32
PITCH = 40

_info = plsc.get_sparse_core_info()
_NC, _NS = _info.num_cores, _info.num_subcores
_NW = _NC * _NS


def _make_gather(B: int, D: int, chunk: int, nbuf: int = 4, dist: int = 2):
    assert B % (_NW * chunk) == 0
    b_per_w = B // _NW
    n_chunks = b_per_w // chunk
    assert n_chunks % nbuf == 0 and dist < nbuf
    mesh = plsc.VectorSubcoreMesh(core_axis_name="c", subcore_axis_name="s")

    @functools.partial(
        pl.kernel,
        out_type=jax.ShapeDtypeStruct((B, D), jnp.float32),
        mesh=mesh,
        scratch_types=[
            pltpu.VMEM((b_per_w,), jnp.int32),
            *[pltpu.VMEM((chunk, D), jnp.float32) for _ in range(nbuf)],
            *[pltpu.SemaphoreType.DMA for _ in range(2 * nbuf)],
        ],
        compiler_params=pltpu.CompilerParams(use_tc_tiling_on_sc=False),
    )
    def gather_kernel(idx_hbm, table_hbm, out_hbm, idx_v, *scratch):
        rows = scratch[:nbuf]
        gsem = scratch[nbuf : 2 * nbuf]
        osem = scratch[2 * nbuf :]
        wid = lax.axis_index("s") * _NC + lax.axis_index("c")
        base = wid * b_per_w
        pltpu.sync_copy(idx_hbm.at[pl.ds(base, b_per_w)], idx_v)

        def g_start(i, b):
            pltpu.async_copy(
                table_hbm.at[idx_v.at[pl.ds(i * chunk, chunk)]], rows[b], gsem[b]
            )

        def g_wait(i, b):
            pltpu.make_async_copy(
                table_hbm.at[idx_v.at[pl.ds(i * chunk, chunk)]], rows[b], gsem[b]
            ).wait()

        def o_start(i, b):
            pltpu.async_copy(
                rows[b], out_hbm.at[pl.ds(base + i * chunk, chunk)], osem[b]
            )

        def o_wait(i, b):
            pltpu.make_async_copy(
                rows[b], out_hbm.at[pl.ds(base + i * chunk, chunk)], osem[b]
            ).wait()

        for b in range(dist):
            g_start(b, b)

        def body(g, carry):
            for b in range(nbuf):
                i = g * nbuf + b
                g_wait(i, b)
                o_start(i, b)
                nxt = i + dist
                c = (b + dist) % nbuf

                @pl.when(nxt < n_chunks)
                def _():
                    @pl.when(nxt >= nbuf)
                    def _():
                        o_wait(nxt - nbuf, c)

                    g_start(nxt, c)

            return carry

        lax.fori_loop(0, n_chunks // nbuf, body, 0)
        for b in range(nbuf):
            o_wait(n_chunks - nbuf + b, b)

    return gather_kernel


def _make_detile(V: int, D: int, n: int, m: int):
    assert D == 32 and V == 100000 and n == 16384 and m == 50
    full_tiles = V // 128
    rem = V - full_tiles * 128
    per_w = full_tiles // _NW + 1
    xL = 4096
    x_items_full = (m // 8) * (n // xL)
    x_rem_rows = m - (m // 8) * 8
    mesh = plsc.VectorSubcoreMesh(core_axis_name="c", subcore_axis_name="s")

    @functools.partial(
        pl.kernel,
        out_type=[
            jax.ShapeDtypeStruct((V * PITCH,), jnp.float32),
            jax.ShapeDtypeStruct((m * n,), jnp.int32),
        ],
        mesh=mesh,
        scratch_types=[
            pltpu.VMEM((D, 128), jnp.float32),
            pltpu.VMEM((D, rem), jnp.float32),
            pltpu.VMEM((128 * PITCH,), jnp.float32),
            pltpu.VMEM((rem * PITCH,), jnp.float32),
            pltpu.VMEM((8, xL), jnp.int32),
            pltpu.VMEM((x_rem_rows, xL), jnp.int32),
        ],
        compiler_params=pltpu.CompilerParams(
            use_tc_tiling_on_sc=True, needs_layout_passes=False
        ),
    )
    def detile_kernel(tab_hbm, x_hbm, tab_out, idx_out, tblk, tblk_p, rowbuf, rowbuf_p, xblk8, xblk2):
        wid = lax.axis_index("s") * _NC + lax.axis_index("c")
        iota = lax.iota(jnp.int32, 16)
        P = [(iota + l0) * PITCH for l0 in range(0, 128, 16)]

        def detile_block(src, dst, width):
            def erow(e, carry):
                for li in range(width // 16):
                    vals = src[e, pl.ds(li * 16, 16)]
                    plsc.store_scatter(dst, [P[li] + e], vals)
                return carry

            lax.fori_loop(0, D, erow, 0)

        def tblock(t, carry):
            k = wid + t * _NW

            @pl.when(k < full_tiles)
            def _():
                pltpu.sync_copy(tab_hbm.at[:, pl.ds(k * 128, 128)], tblk)
                detile_block(tblk, rowbuf, 128)
                pltpu.sync_copy(
                    rowbuf, tab_out.at[pl.ds(k * 128 * PITCH, 128 * PITCH)]
                )

            return carry

        lax.fori_loop(0, per_w, tblock, 0)

        @pl.when(wid == full_tiles % _NW)
        def _():
            v0 = full_tiles * 128
            pltpu.sync_copy(tab_hbm.at[:, pl.ds(v0, rem)], tblk_p)
            detile_block(tblk_p, rowbuf_p, rem)
            pltpu.sync_copy(rowbuf_p, tab_out.at[pl.ds(v0 * PITCH, rem * PITCH)])

        @pl.when(wid < x_items_full)
        def _():
            jr = wid // (n // xL)
            tcb = wid % (n // xL)
            pltpu.sync_copy(
                x_hbm.at[pl.ds(jr * 8, 8), pl.ds(tcb * xL, xL)], xblk8
            )
            for s in range(8):
                pltpu.sync_copy(
                    xblk8.at[s],
                    idx_out.at[pl.ds((jr * 8 + s) * n + tcb * xL, xL)],
                )

        @pl.when((wid >= x_items_full) & (wid < x_items_full + n // xL))
        def _():
            tcb = wid - x_items_full
            j0 = (m // 8) * 8
            pltpu.sync_copy(
                x_hbm.at[pl.ds(j0, x_rem_rows), pl.ds(tcb * xL, xL)], xblk2
            )
            for s in range(x_rem_rows):
                pltpu.sync_copy(
                    xblk2.at[s],
                    idx_out.at[pl.ds((j0 + s) * n + tcb * xL, xL)],
                )

    return detile_kernel


def _make_retile(m: int, n: int, D: int):
    IB = 512
    assert n == _NW * IB and D % 8 == 0
    ntr = D // 8
    mesh = plsc.VectorSubcoreMesh(core_axis_name="c", subcore_axis_name="s")

    @functools.partial(
        pl.kernel,
        out_type=jax.ShapeDtypeStruct((m, D, n), jnp.float32),
        mesh=mesh,
        scratch_types=[
            *[pltpu.VMEM((IB * PITCH,), jnp.float32) for _ in range(2)],
            *[pltpu.VMEM((ntr, 8, IB), jnp.float32) for _ in range(2)],
            *[pltpu.SemaphoreType.DMA for _ in range(4)],
        ],
        compiler_params=pltpu.CompilerParams(
            use_tc_tiling_on_sc=True, needs_layout_passes=False
        ),
    )
    def retile_kernel(in_hbm, out_hbm, inb0, inb1, tb0, tb1, is0, is1, os0, os1):
        inb = (inb0, inb1)
        tb = (tb0, tb1)
        isem = (is0, is1)
        osem = (os0, os1)
        wid = lax.axis_index("s") * _NC + lax.axis_index("c")
        iota = lax.iota(jnp.int32, 16)

        def in_off(j):
            return (j * n + wid * IB) * PITCH

        def i_start(j, p):
            pltpu.async_copy(in_hbm.at[pl.ds(in_off(j), IB * PITCH)], inb[p], isem[p])

        def i_wait(j, p):
            pltpu.make_async_copy(
                in_hbm.at[pl.ds(in_off(j), IB * PITCH)], inb[p], isem[p]
            ).wait()

        def o_start(j, p):
            for tr in range(ntr):
                pltpu.async_copy(
                    tb[p].at[tr],
                    out_hbm.at[j, pl.ds(tr * 8, 8), pl.ds(wid * IB, IB)],
                    osem[p],
                )

        def o_wait(j, p):
            for tr in range(ntr):
                pltpu.make_async_copy(
                    tb[p].at[tr],
                    out_hbm.at[j, pl.ds(tr * 8, 8), pl.ds(wid * IB, IB)],
                    osem[p],
                ).wait()

        i_start(0, 0)

        def body(jg, carry):
            for p in range(2):
                j = 2 * jg + p
                i_wait(j, p)

                @pl.when(j + 1 < m)
                def _():
                    i_start(j + 1, 1 - p)

                @pl.when(j >= 2)
                def _():
                    o_wait(j - 2, p)

                @plsc.parallel_loop(0, IB // 16, unroll=4)
                def lgroup(l0g):
                    src = (iota + l0g * 16) * PITCH
                    l0 = l0g * 16
                    for e in range(D):
                        vals = plsc.load_gather(inb[p], [src + e])
                        tb[p][e // 8, e % 8, pl.ds(l0, 16)] = vals

                o_start(j, p)

            return carry

        lax.fori_loop(0, m // 2, body, 0)
        o_wait(m - 2, 0)
        o_wait(m - 1, 1)

    return retile_kernel


def kernel(x, atom_emb_weight):
    n, m = x.shape
    B = x.size
    V, D = atom_emb_weight.shape
    tab1d, idx1d = _make_detile(V, D, n, m)(
        atom_emb_weight.T, x.T.astype(jnp.int32)
    )
    out = _make_gather(B, PITCH, 640)(idx1d, tab1d.reshape(V, PITCH))
    out_t = _make_retile(m, n, D)(out.reshape(-1))
    return out_t.transpose(2, 0, 1)


# speedup vs baseline: 3.4558x; 1.0757x over previous
import functools

import jax
import jax.numpy as jnp
from jax import lax
from jax.experimental import pallas as pl
from jax.experimental.pallas import tpu as pltpu
from jax.experimental.pallas import tpu_sc as plsc

EMB_SIZE = 32
PITCH = 40

_info = plsc.get_sparse_core_info()
_NC, _NS = _info.num_cores, _info.num_subcores
_NW = _NC * _NS


def _make_gather(B: int, D: int, chunk: int, nbuf: int = 4, dist: int = 2):
    assert B % (_NW * chunk) == 0
    b_per_w = B // _NW
    n_chunks = b_per_w // chunk
    assert n_chunks % nbuf == 0 and dist < nbuf
    mesh = plsc.VectorSubcoreMesh(core_axis_name="c", subcore_axis_name="s")

    @functools.partial(
        pl.kernel,
        out_type=jax.ShapeDtypeStruct((B, D), jnp.float32),
        mesh=mesh,
        scratch_types=[
            pltpu.VMEM((b_per_w,), jnp.int32),
            *[pltpu.VMEM((chunk, D), jnp.float32) for _ in range(nbuf)],
            *[pltpu.SemaphoreType.DMA for _ in range(2 * nbuf)],
        ],
        compiler_params=pltpu.CompilerParams(use_tc_tiling_on_sc=False),
    )
    def gather_kernel(idx_hbm, table_hbm, out_hbm, idx_v, *scratch):
        rows = scratch[:nbuf]
        gsem = scratch[nbuf : 2 * nbuf]
        osem = scratch[2 * nbuf :]
        wid = lax.axis_index("s") * _NC + lax.axis_index("c")
        base = wid * b_per_w
        pltpu.sync_copy(idx_hbm.at[pl.ds(base, b_per_w)], idx_v)

        def g_start(i, b):
            pltpu.async_copy(
                table_hbm.at[idx_v.at[pl.ds(i * chunk, chunk)]], rows[b], gsem[b]
            )

        def g_wait(i, b):
            pltpu.make_async_copy(
                table_hbm.at[idx_v.at[pl.ds(i * chunk, chunk)]], rows[b], gsem[b]
            ).wait()

        def o_start(i, b):
            pltpu.async_copy(
                rows[b], out_hbm.at[pl.ds(base + i * chunk, chunk)], osem[b]
            )

        def o_wait(i, b):
            pltpu.make_async_copy(
                rows[b], out_hbm.at[pl.ds(base + i * chunk, chunk)], osem[b]
            ).wait()

        for b in range(dist):
            g_start(b, b)

        def body(g, carry):
            for b in range(nbuf):
                i = g * nbuf + b
                g_wait(i, b)
                o_start(i, b)
                nxt = i + dist
                c = (b + dist) % nbuf

                @pl.when(nxt < n_chunks)
                def _():
                    @pl.when(nxt >= nbuf)
                    def _():
                        o_wait(nxt - nbuf, c)

                    g_start(nxt, c)

            return carry

        lax.fori_loop(0, n_chunks // nbuf, body, 0)
        for b in range(nbuf):
            o_wait(n_chunks - nbuf + b, b)

    return gather_kernel


def _make_detile(V: int, D: int, n: int, m: int):
    assert D == 32 and V == 100000 and n == 16384 and m == 50
    full_tiles = V // 128
    rem = V - full_tiles * 128
    per_w = full_tiles // _NW + 1
    xL = 4096
    x_items_full = (m // 8) * (n // xL)
    x_rem_rows = m - (m // 8) * 8
    mesh = plsc.VectorSubcoreMesh(core_axis_name="c", subcore_axis_name="s")

    @functools.partial(
        pl.kernel,
        out_type=[
            jax.ShapeDtypeStruct((V * PITCH,), jnp.float32),
            jax.ShapeDtypeStruct((m * n,), jnp.int32),
        ],
        mesh=mesh,
        scratch_types=[
            pltpu.VMEM((D, 128), jnp.float32),
            pltpu.VMEM((D, rem), jnp.float32),
            pltpu.VMEM((128 * PITCH,), jnp.float32),
            pltpu.VMEM((rem * PITCH,), jnp.float32),
            pltpu.VMEM((8, xL), jnp.int32),
            pltpu.VMEM((x_rem_rows, xL), jnp.int32),
        ],
        compiler_params=pltpu.CompilerParams(
            use_tc_tiling_on_sc=True, needs_layout_passes=False
        ),
    )
    def detile_kernel(tab_hbm, x_hbm, tab_out, idx_out, tblk, tblk_p, rowbuf, rowbuf_p, xblk8, xblk2):
        wid = lax.axis_index("s") * _NC + lax.axis_index("c")
        iota = lax.iota(jnp.int32, 16)
        P = [(iota + l0) * PITCH for l0 in range(0, 128, 16)]

        def detile_block(src, dst, width):
            def erow(e, carry):
                for li in range(width // 16):
                    vals = src[e, pl.ds(li * 16, 16)]
                    plsc.store_scatter(dst, [P[li] + e], vals)
                return carry

            lax.fori_loop(0, D, erow, 0)

        def tblock(t, carry):
            k = wid + t * _NW

            @pl.when(k < full_tiles)
            def _():
                pltpu.sync_copy(tab_hbm.at[:, pl.ds(k * 128, 128)], tblk)
                detile_block(tblk, rowbuf, 128)
                pltpu.sync_copy(
                    rowbuf, tab_out.at[pl.ds(k * 128 * PITCH, 128 * PITCH)]
                )

            return carry

        lax.fori_loop(0, per_w, tblock, 0)

        @pl.when(wid == full_tiles % _NW)
        def _():
            v0 = full_tiles * 128
            pltpu.sync_copy(tab_hbm.at[:, pl.ds(v0, rem)], tblk_p)
            detile_block(tblk_p, rowbuf_p, rem)
            pltpu.sync_copy(rowbuf_p, tab_out.at[pl.ds(v0 * PITCH, rem * PITCH)])

        @pl.when(wid < x_items_full)
        def _():
            jr = wid // (n // xL)
            tcb = wid % (n // xL)
            pltpu.sync_copy(
                x_hbm.at[pl.ds(jr * 8, 8), pl.ds(tcb * xL, xL)], xblk8
            )
            for s in range(8):
                pltpu.sync_copy(
                    xblk8.at[s],
                    idx_out.at[pl.ds((jr * 8 + s) * n + tcb * xL, xL)],
                )

        @pl.when((wid >= x_items_full) & (wid < x_items_full + n // xL))
        def _():
            tcb = wid - x_items_full
            j0 = (m // 8) * 8
            pltpu.sync_copy(
                x_hbm.at[pl.ds(j0, x_rem_rows), pl.ds(tcb * xL, xL)], xblk2
            )
            for s in range(x_rem_rows):
                pltpu.sync_copy(
                    xblk2.at[s],
                    idx_out.at[pl.ds((j0 + s) * n + tcb * xL, xL)],
                )

    return detile_kernel


def _make_retile(m: int, n: int, D: int):
    IB = 512
    assert n == _NW * IB and D % 8 == 0
    ntr = D // 8
    mesh = plsc.VectorSubcoreMesh(core_axis_name="c", subcore_axis_name="s")

    @functools.partial(
        pl.kernel,
        out_type=jax.ShapeDtypeStruct((m, D, n), jnp.float32),
        mesh=mesh,
        scratch_types=[
            *[pltpu.VMEM((IB * PITCH,), jnp.float32) for _ in range(2)],
            *[pltpu.VMEM((ntr, 8, IB), jnp.float32) for _ in range(2)],
            *[pltpu.SemaphoreType.DMA for _ in range(4)],
        ],
        compiler_params=pltpu.CompilerParams(
            use_tc_tiling_on_sc=True, needs_layout_passes=False
        ),
    )
    def retile_kernel(in_hbm, out_hbm, inb0, inb1, tb0, tb1, is0, is1, os0, os1):
        inb = (inb0, inb1)
        tb = (tb0, tb1)
        isem = (is0, is1)
        osem = (os0, os1)
        wid = lax.axis_index("s") * _NC + lax.axis_index("c")
        iota = lax.iota(jnp.int32, 16)

        def in_off(j):
            return (j * n + wid * IB) * PITCH

        def i_start(j, p):
            pltpu.async_copy(in_hbm.at[pl.ds(in_off(j), IB * PITCH)], inb[p], isem[p])

        def i_wait(j, p):
            pltpu.make_async_copy(
                in_hbm.at[pl.ds(in_off(j), IB * PITCH)], inb[p], isem[p]
            ).wait()

        def o_start(j, p):
            for tr in range(ntr):
                pltpu.async_copy(
                    tb[p].at[tr],
                    out_hbm.at[j, pl.ds(tr * 8, 8), pl.ds(wid * IB, IB)],
                    osem[p],
                )

        def o_wait(j, p):
            for tr in range(ntr):
                pltpu.make_async_copy(
                    tb[p].at[tr],
                    out_hbm.at[j, pl.ds(tr * 8, 8), pl.ds(wid * IB, IB)],
                    osem[p],
                ).wait()

        i_start(0, 0)

        def body(jg, carry):
            for p in range(2):
                j = 2 * jg + p
                i_wait(j, p)

                @pl.when(j + 1 < m)
                def _():
                    i_start(j + 1, 1 - p)

                @pl.when(j >= 2)
                def _():
                    o_wait(j - 2, p)

                iotap = iota * PITCH

                @plsc.parallel_loop(0, IB // 16, unroll=8)
                def lgroup(l0g):
                    src = iotap + l0g * (16 * PITCH)
                    l0 = l0g * 16
                    for e in range(D):
                        vals = plsc.load_gather(inb[p], [src + e])
                        tb[p][e // 8, e % 8, pl.ds(l0, 16)] = vals

                o_start(j, p)

            return carry

        lax.fori_loop(0, m // 2, body, 0)
        o_wait(m - 2, 0)
        o_wait(m - 1, 1)

    return retile_kernel


def kernel(x, atom_emb_weight):
    n, m = x.shape
    B = x.size
    V, D = atom_emb_weight.shape
    tab1d, idx1d = _make_detile(V, D, n, m)(
        atom_emb_weight.T, x.T.astype(jnp.int32)
    )
    out = _make_gather(B, PITCH, 640)(idx1d, tab1d.reshape(V, PITCH))
    out_t = _make_retile(m, n, D)(out.reshape(-1))
    return out_t.transpose(2, 0, 1)


# speedup vs baseline: 3.6856x; 1.0665x over previous
import functools

import jax
import jax.numpy as jnp
from jax import lax
from jax.experimental import pallas as pl
from jax.experimental.pallas import tpu as pltpu
from jax.experimental.pallas import tpu_sc as plsc

EMB_SIZE = 32
PITCH = 40

_info = plsc.get_sparse_core_info()
_NC, _NS = _info.num_cores, _info.num_subcores
_NW = _NC * _NS


def _make_gather(B: int, D: int, chunk: int, nbuf: int = 4, dist: int = 2):
    assert B % (_NW * chunk) == 0
    b_per_w = B // _NW
    n_chunks = b_per_w // chunk
    assert n_chunks % nbuf == 0 and dist < nbuf
    mesh = plsc.VectorSubcoreMesh(core_axis_name="c", subcore_axis_name="s")

    @functools.partial(
        pl.kernel,
        out_type=jax.ShapeDtypeStruct((B, D), jnp.float32),
        mesh=mesh,
        scratch_types=[
            pltpu.VMEM((b_per_w,), jnp.int32),
            *[pltpu.VMEM((chunk, D), jnp.float32) for _ in range(nbuf)],
            *[pltpu.SemaphoreType.DMA for _ in range(2 * nbuf)],
        ],
        compiler_params=pltpu.CompilerParams(use_tc_tiling_on_sc=False),
    )
    def gather_kernel(idx_hbm, table_hbm, out_hbm, idx_v, *scratch):
        rows = scratch[:nbuf]
        gsem = scratch[nbuf : 2 * nbuf]
        osem = scratch[2 * nbuf :]
        wid = lax.axis_index("s") * _NC + lax.axis_index("c")
        base = wid * b_per_w
        pltpu.sync_copy(idx_hbm.at[pl.ds(base, b_per_w)], idx_v)

        def g_start(i, b):
            pltpu.async_copy(
                table_hbm.at[idx_v.at[pl.ds(i * chunk, chunk)]], rows[b], gsem[b]
            )

        def g_wait(i, b):
            pltpu.make_async_copy(
                table_hbm.at[idx_v.at[pl.ds(i * chunk, chunk)]], rows[b], gsem[b]
            ).wait()

        def o_start(i, b):
            pltpu.async_copy(
                rows[b], out_hbm.at[pl.ds(base + i * chunk, chunk)], osem[b]
            )

        def o_wait(i, b):
            pltpu.make_async_copy(
                rows[b], out_hbm.at[pl.ds(base + i * chunk, chunk)], osem[b]
            ).wait()

        for b in range(dist):
            g_start(b, b)

        def body(g, carry):
            for b in range(nbuf):
                i = g * nbuf + b
                g_wait(i, b)
                o_start(i, b)
                nxt = i + dist
                c = (b + dist) % nbuf

                @pl.when(nxt < n_chunks)
                def _():
                    @pl.when(nxt >= nbuf)
                    def _():
                        o_wait(nxt - nbuf, c)

                    g_start(nxt, c)

            return carry

        lax.fori_loop(0, n_chunks // nbuf, body, 0)
        for b in range(nbuf):
            o_wait(n_chunks - nbuf + b, b)

    return gather_kernel


def _make_detile(V: int, D: int, n: int, m: int):
    assert D == 32 and V == 100000 and n == 16384 and m == 50
    full_tiles = V // 128
    rem = V - full_tiles * 128
    per_w = full_tiles // _NW + 1
    xL = 4096
    x_items_full = (m // 8) * (n // xL)
    x_rem_rows = m - (m // 8) * 8
    mesh = plsc.VectorSubcoreMesh(core_axis_name="c", subcore_axis_name="s")

    @functools.partial(
        pl.kernel,
        out_type=[
            jax.ShapeDtypeStruct((V * PITCH,), jnp.float32),
            jax.ShapeDtypeStruct((m * n,), jnp.int32),
        ],
        mesh=mesh,
        scratch_types=[
            pltpu.VMEM((D, 128), jnp.float32),
            pltpu.VMEM((D, rem), jnp.float32),
            pltpu.VMEM((128 * PITCH,), jnp.float32),
            pltpu.VMEM((rem * PITCH,), jnp.float32),
            pltpu.VMEM((8, xL), jnp.int32),
            pltpu.VMEM((x_rem_rows, xL), jnp.int32),
        ],
        compiler_params=pltpu.CompilerParams(
            use_tc_tiling_on_sc=True, needs_layout_passes=False
        ),
    )
    def detile_kernel(tab_hbm, x_hbm, tab_out, idx_out, tblk, tblk_p, rowbuf, rowbuf_p, xblk8, xblk2):
        wid = lax.axis_index("s") * _NC + lax.axis_index("c")
        iota = lax.iota(jnp.int32, 16)
        P = [(iota + l0) * PITCH for l0 in range(0, 128, 16)]

        def detile_block(src, dst, width):
            @plsc.parallel_loop(0, D, unroll=8)
            def erow(e):
                for li in range(width // 16):
                    vals = src[e, pl.ds(li * 16, 16)]
                    plsc.store_scatter(dst, [P[li] + e], vals)

        def tblock(t, carry):
            k = wid + t * _NW

            @pl.when(k < full_tiles)
            def _():
                pltpu.sync_copy(tab_hbm.at[:, pl.ds(k * 128, 128)], tblk)
                detile_block(tblk, rowbuf, 128)
                pltpu.sync_copy(
                    rowbuf, tab_out.at[pl.ds(k * 128 * PITCH, 128 * PITCH)]
                )

            return carry

        lax.fori_loop(0, per_w, tblock, 0)

        @pl.when(wid == full_tiles % _NW)
        def _():
            v0 = full_tiles * 128
            pltpu.sync_copy(tab_hbm.at[:, pl.ds(v0, rem)], tblk_p)
            detile_block(tblk_p, rowbuf_p, rem)
            pltpu.sync_copy(rowbuf_p, tab_out.at[pl.ds(v0 * PITCH, rem * PITCH)])

        @pl.when(wid < x_items_full)
        def _():
            jr = wid // (n // xL)
            tcb = wid % (n // xL)
            pltpu.sync_copy(
                x_hbm.at[pl.ds(jr * 8, 8), pl.ds(tcb * xL, xL)], xblk8
            )
            for s in range(8):
                pltpu.sync_copy(
                    xblk8.at[s],
                    idx_out.at[pl.ds((jr * 8 + s) * n + tcb * xL, xL)],
                )

        @pl.when((wid >= x_items_full) & (wid < x_items_full + n // xL))
        def _():
            tcb = wid - x_items_full
            j0 = (m // 8) * 8
            pltpu.sync_copy(
                x_hbm.at[pl.ds(j0, x_rem_rows), pl.ds(tcb * xL, xL)], xblk2
            )
            for s in range(x_rem_rows):
                pltpu.sync_copy(
                    xblk2.at[s],
                    idx_out.at[pl.ds((j0 + s) * n + tcb * xL, xL)],
                )

    return detile_kernel


def _make_retile(m: int, n: int, D: int):
    IB = 512
    assert n == _NW * IB and D % 8 == 0
    ntr = D // 8
    mesh = plsc.VectorSubcoreMesh(core_axis_name="c", subcore_axis_name="s")

    @functools.partial(
        pl.kernel,
        out_type=jax.ShapeDtypeStruct((m, D, n), jnp.float32),
        mesh=mesh,
        scratch_types=[
            *[pltpu.VMEM((IB * PITCH,), jnp.float32) for _ in range(2)],
            *[pltpu.VMEM((ntr, 8, IB), jnp.float32) for _ in range(2)],
            *[pltpu.SemaphoreType.DMA for _ in range(4)],
        ],
        compiler_params=pltpu.CompilerParams(
            use_tc_tiling_on_sc=True, needs_layout_passes=False
        ),
    )
    def retile_kernel(in_hbm, out_hbm, inb0, inb1, tb0, tb1, is0, is1, os0, os1):
        inb = (inb0, inb1)
        tb = (tb0, tb1)
        isem = (is0, is1)
        osem = (os0, os1)
        wid = lax.axis_index("s") * _NC + lax.axis_index("c")
        iota = lax.iota(jnp.int32, 16)

        def in_off(j):
            return (j * n + wid * IB) * PITCH

        def i_start(j, p):
            pltpu.async_copy(in_hbm.at[pl.ds(in_off(j), IB * PITCH)], inb[p], isem[p])

        def i_wait(j, p):
            pltpu.make_async_copy(
                in_hbm.at[pl.ds(in_off(j), IB * PITCH)], inb[p], isem[p]
            ).wait()

        def o_start(j, p):
            for tr in range(ntr):
                pltpu.async_copy(
                    tb[p].at[tr],
                    out_hbm.at[j, pl.ds(tr * 8, 8), pl.ds(wid * IB, IB)],
                    osem[p],
                )

        def o_wait(j, p):
            for tr in range(ntr):
                pltpu.make_async_copy(
                    tb[p].at[tr],
                    out_hbm.at[j, pl.ds(tr * 8, 8), pl.ds(wid * IB, IB)],
                    osem[p],
                ).wait()

        i_start(0, 0)

        def body(jg, carry):
            for p in range(2):
                j = 2 * jg + p
                i_wait(j, p)

                @pl.when(j + 1 < m)
                def _():
                    i_start(j + 1, 1 - p)

                @pl.when(j >= 2)
                def _():
                    o_wait(j - 2, p)

                iotap = iota * PITCH

                @plsc.parallel_loop(0, IB // 16, unroll=8)
                def lgroup(l0g):
                    src = iotap + l0g * (16 * PITCH)
                    l0 = l0g * 16
                    for e in range(D):
                        vals = plsc.load_gather(inb[p], [src + e])
                        tb[p][e // 8, e % 8, pl.ds(l0, 16)] = vals

                o_start(j, p)

            return carry

        lax.fori_loop(0, m // 2, body, 0)
        o_wait(m - 2, 0)
        o_wait(m - 1, 1)

    return retile_kernel


def kernel(x, atom_emb_weight):
    n, m = x.shape
    B = x.size
    V, D = atom_emb_weight.shape
    tab1d, idx1d = _make_detile(V, D, n, m)(
        atom_emb_weight.T, x.T.astype(jnp.int32)
    )
    out = _make_gather(B, PITCH, 640)(idx1d, tab1d.reshape(V, PITCH))
    out_t = _make_retile(m, n, D)(out.reshape(-1))
    return out_t.transpose(2, 0, 1)
